# Initial kernel scaffold; baseline (speedup 1.0000x reference)
import jax
import jax.numpy as jnp
from jax.experimental import pallas as pl


def kernel(input_ids, position_ids, pos_emb):
    raise NotImplementedError("write your pallas kernel here")


# speedup vs baseline: 1.0256x; 1.0256x over previous
import functools

import jax
import jax.numpy as jnp
from jax import lax
from jax.experimental import pallas as pl
from jax.experimental.pallas import tpu as pltpu
from jax.experimental.pallas import tpu_sc as plsc

BATCH = 4
SEQ = 8192
HIDDEN = 1024
ROWS = BATCH * SEQ
LANES = 16

NUM_CORES = 2
NUM_SUBCORES = 16
NW = NUM_CORES * NUM_SUBCORES
ROWS_PER_W = ROWS // NW
CHUNK = 32
N_CHUNKS = ROWS_PER_W // CHUNK


def _body(x_hbm, ids_hbm, emb_hbm, out_hbm, idx_v, in_v, g_v, sem_in, sem_g):
    wid = lax.axis_index("s") * NUM_CORES + lax.axis_index("c")
    base = wid * ROWS_PER_W

    def chunk_body(c, carry):
        off = base + c * CHUNK
        pltpu.sync_copy(ids_hbm.at[pl.ds(off, CHUNK)], idx_v)
        cp_in = pltpu.async_copy(x_hbm.at[pl.ds(off, CHUNK)], in_v, sem_in)
        cp_g = pltpu.async_copy(emb_hbm.at[idx_v], g_v, sem_g)
        cp_in.wait()
        cp_g.wait()

        def row_body(r, carry2):
            for j in range(HIDDEN // LANES):
                sl = pl.ds(j * LANES, LANES)
                in_v[r, sl] = in_v[r, sl] + g_v[r, sl]
            return carry2

        lax.fori_loop(0, CHUNK, row_body, 0)
        pltpu.sync_copy(in_v, out_hbm.at[pl.ds(off, CHUNK)])
        return carry

    lax.fori_loop(0, N_CHUNKS, chunk_body, 0)


@functools.partial(jax.jit, static_argnums=())
def _run(x, ids, emb):
    mesh = plsc.VectorSubcoreMesh(
        core_axis_name="c", subcore_axis_name="s",
        num_cores=NUM_CORES, num_subcores=NUM_SUBCORES)
    f = pl.kernel(
        _body,
        out_type=jax.ShapeDtypeStruct((ROWS, HIDDEN), jnp.float32),
        mesh=mesh,
        scratch_types=[
            pltpu.VMEM((CHUNK,), jnp.int32),
            pltpu.VMEM((CHUNK, HIDDEN), jnp.float32),
            pltpu.VMEM((CHUNK, HIDDEN), jnp.float32),
            pltpu.SemaphoreType.DMA,
            pltpu.SemaphoreType.DMA,
        ],
    )
    return f(x, ids, emb)


def kernel(input_ids, position_ids, pos_emb):
    x = input_ids.reshape(ROWS, HIDDEN)
    ids = position_ids.reshape(ROWS)
    out = _run(x, ids, pos_emb)
    return out.reshape(input_ids.shape)


# speedup vs baseline: 1.4812x; 1.4442x over previous
import functools

import jax
import jax.numpy as jnp
from jax import lax
from jax.experimental import pallas as pl
from jax.experimental.pallas import tpu as pltpu
from jax.experimental.pallas import tpu_sc as plsc

BATCH = 4
SEQ = 8192
HIDDEN = 1024
ROWS = BATCH * SEQ
LANES = 16

NUM_CORES = 2
NUM_SUBCORES = 16
NW = NUM_CORES * NUM_SUBCORES
ROWS_PER_W = ROWS // NW
CHUNK = 16
N_CHUNKS = ROWS_PER_W // CHUNK
NBUF = 2


def _body(x_hbm, ids_hbm, emb_hbm, out_hbm,
          idx_all, in_v, g_v, o_v, sem_in, sem_g, sem_out):
    wid = lax.axis_index("s") * NUM_CORES + lax.axis_index("c")
    base = wid * ROWS_PER_W

    pltpu.sync_copy(ids_hbm.at[pl.ds(base, ROWS_PER_W)], idx_all)

    def start_in(g, b):
        off = base + g * CHUNK
        pltpu.async_copy(x_hbm.at[pl.ds(off, CHUNK)], in_v.at[b], sem_in.at[b])
        pltpu.async_copy(emb_hbm.at[idx_all.at[pl.ds(g * CHUNK, CHUNK)]],
                         g_v.at[b], sem_g.at[b])

    def wait_in(b):
        pltpu.make_async_copy(x_hbm.at[pl.ds(0, CHUNK)], in_v.at[b],
                              sem_in.at[b]).wait()
        pltpu.make_async_copy(x_hbm.at[pl.ds(0, CHUNK)], g_v.at[b],
                              sem_g.at[b]).wait()

    def start_out(g, b):
        off = base + g * CHUNK
        pltpu.async_copy(o_v.at[b], out_hbm.at[pl.ds(off, CHUNK)],
                         sem_out.at[b])

    def wait_out(b):
        pltpu.make_async_copy(o_v.at[b], out_hbm.at[pl.ds(0, CHUNK)],
                              sem_out.at[b]).wait()

    for b in range(NBUF):
        start_in(b, b)

    def super_body(g2, carry):
        for b in range(NBUF):
            g = g2 * NBUF + b
            wait_in(b)

            def row_body(r, carry2):
                for j in range(HIDDEN // LANES):
                    sl = pl.ds(j * LANES, LANES)
                    o_v[b, r, sl] = in_v[b, r, sl] + g_v[b, r, sl]
                return carry2

            lax.fori_loop(0, CHUNK, row_body, 0)

            @pl.when(g >= NBUF)
            def _():
                wait_out(b)

            start_out(g, b)

            @pl.when(g + NBUF < N_CHUNKS)
            def _():
                start_in(g + NBUF, b)
        return carry

    lax.fori_loop(0, N_CHUNKS // NBUF, super_body, 0)

    for b in range(NBUF):
        wait_out(b)


@functools.partial(jax.jit, static_argnums=())
def _run(x, ids, emb):
    mesh = plsc.VectorSubcoreMesh(
        core_axis_name="c", subcore_axis_name="s",
        num_cores=NUM_CORES, num_subcores=NUM_SUBCORES)
    f = pl.kernel(
        _body,
        out_type=jax.ShapeDtypeStruct((ROWS, HIDDEN), jnp.float32),
        mesh=mesh,
        scratch_types=[
            pltpu.VMEM((ROWS_PER_W,), jnp.int32),
            pltpu.VMEM((NBUF, CHUNK, HIDDEN), jnp.float32),
            pltpu.VMEM((NBUF, CHUNK, HIDDEN), jnp.float32),
            pltpu.VMEM((NBUF, CHUNK, HIDDEN), jnp.float32),
            pltpu.SemaphoreType.DMA((NBUF,)),
            pltpu.SemaphoreType.DMA((NBUF,)),
            pltpu.SemaphoreType.DMA((NBUF,)),
        ],
    )
    return f(x, ids, emb)


def kernel(input_ids, position_ids, pos_emb):
    x = input_ids.reshape(ROWS, HIDDEN)
    ids = position_ids.reshape(ROWS)
    out = _run(x, ids, pos_emb)
    return out.reshape(input_ids.shape)


# speedup vs baseline: 1.8183x; 1.2276x over previous
import functools

import jax
import jax.numpy as jnp
from jax import lax
from jax.experimental import pallas as pl
from jax.experimental.pallas import tpu as pltpu
from jax.experimental.pallas import tpu_sc as plsc

BATCH = 4
SEQ = 8192
HIDDEN = 1024
ROWS = BATCH * SEQ
LANES = 16

NUM_CORES = 2
NUM_SUBCORES = 16
NW = NUM_CORES * NUM_SUBCORES
ROWS_PER_W = ROWS // NW
CHUNK = 8
N_CHUNKS = ROWS_PER_W // CHUNK
NBUF = 4
GLEAD = 2


def _body(x_hbm, ids_hbm, emb_hbm, out_hbm,
          idx_all, in_v, g_v, sem_in, sem_g, sem_out):
    wid = lax.axis_index("s") * NUM_CORES + lax.axis_index("c")
    base = wid * ROWS_PER_W

    pltpu.sync_copy(ids_hbm.at[pl.ds(base, ROWS_PER_W)], idx_all)

    def start_lin(g, b):
        pltpu.async_copy(x_hbm.at[pl.ds(base + g * CHUNK, CHUNK)],
                         in_v.at[b], sem_in.at[b])

    def start_gat(g, b):
        pltpu.async_copy(emb_hbm.at[idx_all.at[pl.ds(g * CHUNK, CHUNK)]],
                         g_v.at[b], sem_g.at[b])

    def start_out(g, b):
        pltpu.async_copy(g_v.at[b], out_hbm.at[pl.ds(base + g * CHUNK, CHUNK)],
                         sem_out.at[b])

    def wait_lin(b):
        pltpu.make_async_copy(x_hbm.at[pl.ds(0, CHUNK)], in_v.at[b],
                              sem_in.at[b]).wait()

    def wait_gat(b):
        pltpu.make_async_copy(x_hbm.at[pl.ds(0, CHUNK)], g_v.at[b],
                              sem_g.at[b]).wait()

    def wait_out(b):
        pltpu.make_async_copy(g_v.at[b], out_hbm.at[pl.ds(0, CHUNK)],
                              sem_out.at[b]).wait()

    for b in range(NBUF):
        start_lin(b, b)
        start_gat(b, b)

    def super_body(h2, carry):
        for b in range(NBUF):
            h = h2 * NBUF + b
            wait_gat(b)
            wait_lin(b)

            def row_body(r, carry2):
                for j in range(HIDDEN // LANES):
                    sl = pl.ds(j * LANES, LANES)
                    plsc.addupdate(g_v.at[b, r, sl], in_v[b, r, sl])
                return carry2

            lax.fori_loop(0, CHUNK, row_body, 0)

            start_out(h, b)

            @pl.when(h + NBUF < N_CHUNKS)
            def _():
                start_lin(h + NBUF, b)

            bn = (b + GLEAD) % NBUF

            @pl.when(jnp.logical_and(h + GLEAD >= NBUF,
                                     h + GLEAD < N_CHUNKS))
            def _():
                wait_out(bn)
                start_gat(h + GLEAD, bn)
        return carry

    lax.fori_loop(0, N_CHUNKS // NBUF, super_body, 0)

    for b in range(NBUF):
        wait_out(b)


@functools.partial(jax.jit, static_argnums=())
def _run(x, ids, emb):
    mesh = plsc.VectorSubcoreMesh(
        core_axis_name="c", subcore_axis_name="s",
        num_cores=NUM_CORES, num_subcores=NUM_SUBCORES)
    f = pl.kernel(
        _body,
        out_type=jax.ShapeDtypeStruct((ROWS, HIDDEN), jnp.float32),
        mesh=mesh,
        scratch_types=[
            pltpu.VMEM((ROWS_PER_W,), jnp.int32),
            pltpu.VMEM((NBUF, CHUNK, HIDDEN), jnp.float32),
            pltpu.VMEM((NBUF, CHUNK, HIDDEN), jnp.float32),
            pltpu.SemaphoreType.DMA((NBUF,)),
            pltpu.SemaphoreType.DMA((NBUF,)),
            pltpu.SemaphoreType.DMA((NBUF,)),
        ],
    )
    return f(x, ids, emb)


def kernel(input_ids, position_ids, pos_emb):
    x = input_ids.reshape(ROWS, HIDDEN)
    ids = position_ids.reshape(ROWS)
    out = _run(x, ids, pos_emb)
    return out.reshape(input_ids.shape)
